# baseline (speedup 1.0000x reference)
import functools

import jax
import jax.numpy as jnp
from jax import lax
from jax.experimental import pallas as pl
from jax.experimental.pallas import tpu as pltpu
from jax.experimental.pallas import tpu_sc as plsc

N = 10000
E = 320000
B = 20
NP = 500
KP = 400
D = 128
C = 2
K = B * KP
EC = 128
NSUB = 16
RPT = 162
ERP = RPT * NSUB
EPAD = ERP * EC
NPAD = N + 8
KPAD = K + 8
NBUF = 3

_MESH = plsc.VectorSubcoreMesh(core_axis_name="c", subcore_axis_name="s")


def _run_stream(table, s1d, d1d, acc, sidx, didx, sorig, dorig, rows, gsem,
                ssem, ninv_v, sub):
  base = sub * RPT

  def load_idx(i, p):
    off = (base + i) * EC
    if ninv_v is None:
      pltpu.sync_copy(s1d.at[pl.ds(off, EC)], sidx.at[p])
      pltpu.sync_copy(d1d.at[pl.ds(off, EC)], didx.at[p])
    else:
      pltpu.sync_copy(s1d.at[pl.ds(off, EC)], sorig.at[p])
      pltpu.sync_copy(d1d.at[pl.ds(off, EC)], dorig.at[p])
      lane = lax.iota(jnp.int32, 16)
      for g in range(EC // 16):
        sl = pl.ds(g * 16, 16)
        sv = plsc.load_gather(ninv_v, [sorig[p, sl]])
        dv = plsc.load_gather(ninv_v, [dorig[p, sl]])
        ok = (sv < K) & (dv < K)
        dump = (sub * EC + g * 16) * 3 + lane * 3
        sidx[p, sl] = jnp.where(ok, sv, K)
        didx[p, sl] = jnp.where(ok, dv, dump)

  def gather(p):
    pltpu.async_copy(table.at[sidx.at[p]], rows.at[p], gsem.at[p])

  def wait_gather(p):
    pltpu.make_async_copy(table.at[sidx.at[p]], rows.at[p], gsem.at[p]).wait()

  def scat(p):
    pltpu.async_copy(rows.at[p], acc.at[didx.at[p]], ssem.at[p], add=True)

  def wait_scat(p):
    pltpu.make_async_copy(rows.at[p], acc.at[didx.at[p]], ssem.at[p]).wait()

  for i in range(NBUF):
    load_idx(i, i)
    gather(i)
  wait_gather(0)
  scat(0)
  wait_gather(1)
  scat(1)

  def body3(j, carry):
    for u in range(3):
      i = 3 * j + 3 + u
      p = u
      q = (u + 2) % 3
      wait_scat(p)
      load_idx(i, p)
      gather(p)
      wait_gather(q)
      scat(q)
    return carry

  lax.fori_loop(0, (RPT - 3) // 3, body3, 0)

  wait_gather((RPT - 1) % 3)
  scat((RPT - 1) % 3)
  for p in range(NBUF):
    wait_scat(p)


def _zero_acc(zeros, acc, sub, per, pad_rows):
  pltpu.sync_copy(zeros.at[pl.ds(sub * per, per)],
                  acc.at[pl.ds(sub * per, per)])
  extra = pad_rows - NSUB * per

  @pl.when(sub == 0)
  def _():
    pltpu.sync_copy(zeros.at[pl.ds(0, extra)],
                    acc.at[pl.ds(NSUB * per, extra)])


def _readout(acc, out, sub, per, nrows):
  pltpu.sync_copy(acc.at[pl.ds(sub * per, per)],
                  out.at[pl.ds(sub * per, per)])
  extra = nrows - NSUB * per

  @pl.when(sub == 0)
  def _():
    pltpu.sync_copy(acc.at[pl.ds(NSUB * per, extra)],
                    out.at[pl.ds(NSUB * per, extra)])


def _seg1_body(x1, x2, s1, d1, s2, d2, zeros, agg1, agg2,
               acc, sidx, didx, rows, gsem, ssem):
  c = lax.axis_index("c")
  s = lax.axis_index("s")
  _zero_acc(zeros, acc, s, 624, NPAD)
  plsc.subcore_barrier()

  @pl.when(c == 0)
  def _():
    _run_stream(x1, s1, d1, acc, sidx, didx, None, None, rows, gsem, ssem,
                None, s)

  @pl.when(c == 1)
  def _():
    _run_stream(x2, s2, d2, acc, sidx, didx, None, None, rows, gsem, ssem,
                None, s)

  plsc.subcore_barrier()

  @pl.when(c == 0)
  def _():
    _readout(acc, agg1, s, 624, N)

  @pl.when(c == 1)
  def _():
    _readout(acc, agg2, s, 624, N)


def _seg2_body(t1, t2, s1, d1, s2, d2, ninv, zeros, agg3, agg4,
               acc, sidx, didx, sorig, dorig, rows, ninv_v, gsem, ssem):
  c = lax.axis_index("c")
  s = lax.axis_index("s")
  pltpu.sync_copy(ninv, ninv_v)
  _zero_acc(zeros, acc, s, 496, KPAD)
  plsc.subcore_barrier()

  @pl.when(c == 0)
  def _():
    _run_stream(t1, s1, d1, acc, sidx, didx, sorig, dorig, rows, gsem, ssem,
                ninv_v, s)

  @pl.when(c == 1)
  def _():
    _run_stream(t2, s2, d2, acc, sidx, didx, sorig, dorig, rows, gsem, ssem,
                ninv_v, s)

  plsc.subcore_barrier()

  @pl.when(c == 0)
  def _():
    _readout(acc, agg3, s, 496, K)

  @pl.when(c == 1)
  def _():
    _readout(acc, agg4, s, 496, K)


_NCHUNK = K // 80


def _build_body(h1s, h2s, perm, zeros, h1p, h2p, pidx, prow, psem):
  c = lax.axis_index("c")
  s = lax.axis_index("s")

  def run(table, outp):
    cs = (s * _NCHUNK) // NSUB
    ce = ((s + 1) * _NCHUNK) // NSUB
    for t in range(_NCHUNK // NSUB + 1):
      idx = cs + t

      @pl.when(idx < ce)
      def _():
        q0 = idx * 80
        pltpu.sync_copy(perm.at[pl.ds(q0, 80)], pidx)
        pltpu.async_copy(table.at[pidx], prow, psem).wait()
        pltpu.sync_copy(prow, outp.at[pl.ds(q0, 80)])

    @pl.when(s == 0)
    def _():
      pltpu.sync_copy(zeros.at[pl.ds(0, 8)], outp.at[pl.ds(K, 8)])

  @pl.when(c == 0)
  def _():
    run(h1s, h1p)

  @pl.when(c == 1)
  def _():
    run(h2s, h2p)


_sc_seg1 = functools.partial(
    pl.kernel, _seg1_body,
    out_type=(jax.ShapeDtypeStruct((N, D), jnp.float32),
              jax.ShapeDtypeStruct((N, D), jnp.float32)),
    mesh=_MESH,
    compiler_params=pltpu.CompilerParams(needs_layout_passes=False),
    scratch_types=(
        pltpu.VMEM_SHARED((NPAD, D), jnp.float32),
        pltpu.VMEM((NBUF, EC), jnp.int32),
        pltpu.VMEM((NBUF, EC), jnp.int32),
        pltpu.VMEM((NBUF, EC, D), jnp.float32),
        pltpu.SemaphoreType.DMA((NBUF,)),
        pltpu.SemaphoreType.DMA((NBUF,)),
    ))()

_sc_seg2 = functools.partial(
    pl.kernel, _seg2_body,
    out_type=(jax.ShapeDtypeStruct((K, D), jnp.float32),
              jax.ShapeDtypeStruct((K, D), jnp.float32)),
    mesh=_MESH,
    compiler_params=pltpu.CompilerParams(needs_layout_passes=False),
    scratch_types=(
        pltpu.VMEM_SHARED((KPAD, D), jnp.float32),
        pltpu.VMEM((NBUF, EC), jnp.int32),
        pltpu.VMEM((NBUF, EC), jnp.int32),
        pltpu.VMEM((NBUF, EC), jnp.int32),
        pltpu.VMEM((NBUF, EC), jnp.int32),
        pltpu.VMEM((NBUF, EC, D), jnp.float32),
        pltpu.VMEM((NPAD,), jnp.int32),
        pltpu.SemaphoreType.DMA((NBUF,)),
        pltpu.SemaphoreType.DMA((NBUF,)),
    ))()

_sc_build = functools.partial(
    pl.kernel, _build_body,
    out_type=(jax.ShapeDtypeStruct((KPAD, D), jnp.float32),
              jax.ShapeDtypeStruct((KPAD, D), jnp.float32)),
    mesh=_MESH,
    compiler_params=pltpu.CompilerParams(needs_layout_passes=False),
    scratch_types=(
        pltpu.VMEM((80,), jnp.int32),
        pltpu.VMEM((80, D), jnp.float32),
        pltpu.SemaphoreType.DMA,
    ))()


_HI = lax.Precision.HIGHEST


def _mm(a, b):
  return jnp.matmul(a, b)


def _bnrelu(t, g, bt):
  m = jnp.mean(t, axis=0, keepdims=True)
  v = jnp.mean((t - m) * (t - m), axis=0, keepdims=True)
  return jnp.maximum((t - m) * lax.rsqrt(v + 1e-5) * g + bt, 0.0)


def _tc1_body(x, fcx, agg1, agg2, wr1, wo1, b1, g1, bt1, wr2, wo2, b2, g2,
              bt2, pw, h1s_o, h2s_o, ninv_o, perm_o):
  xx = x[...]
  ff = fcx[...]
  h1 = _bnrelu(_mm(agg1[...], wr1[...]) + _mm(xx, wo1[...]) + b1[...], g1[...],
               bt1[...])
  h2 = _bnrelu(_mm(agg2[...], wr2[...]) + _mm(ff, wo2[...]) + b2[...], g2[...],
               bt2[...])
  pwv = pw[...]
  score = ((_mm(h1, pwv[:D]) + _mm(h2, pwv[D:]))
           * lax.rsqrt(jnp.sum(pwv * pwv)))
  sc = jnp.tanh(score)
  h1s_o[...] = h1 * sc
  h2s_o[...] = h2 * sc

  ii = lax.broadcasted_iota(jnp.int32, (NP, NP), 0)
  jj = lax.broadcasted_iota(jnp.int32, (NP, NP), 1)
  eye = jnp.where(ii == jj, 1.0, 0.0)
  jlt = jj < ii
  kio = lax.broadcasted_iota(jnp.int32, (KP, NP), 0).astype(jnp.float32)
  icol = lax.broadcasted_iota(jnp.int32, (NP, 1), 0).astype(jnp.float32)
  tdims = (((0,), (0,)), ((), ()))
  hi = lax.Precision.HIGHEST
  for g in range(B):
    sg = lax.slice(score, (g * NP, 0), ((g + 1) * NP, 1))
    sgr = lax.dot_general(sg, eye, tdims, precision=hi)
    beats = (sgr > sg) | ((sgr == sg) & jlt)
    rank = jnp.sum(jnp.where(beats, 1.0, 0.0), axis=1, keepdims=True)
    nn = jnp.where(rank < KP, rank + g * KP, float(K))
    ninv_o[pl.ds(g * NP, NP), :] = nn.astype(jnp.int32)
    rr = lax.dot_general(rank, eye, tdims, precision=hi)
    oh = jnp.where(rr == kio, 1.0, 0.0)
    loc = lax.dot_general(oh, icol, (((1,), (0,)), ((), ())), precision=hi)
    perm_o[pl.ds(g * KP, KP), :] = (loc + g * NP).astype(jnp.int32)
  ninv_o[pl.ds(N, 8), :] = jnp.full((8, 1), K, jnp.int32)


_tc1 = pl.pallas_call(
    _tc1_body,
    out_shape=(jax.ShapeDtypeStruct((N, D), jnp.float32),
               jax.ShapeDtypeStruct((N, D), jnp.float32),
               jax.ShapeDtypeStruct((NPAD, 1), jnp.int32),
               jax.ShapeDtypeStruct((K, 1), jnp.int32)))


def _tc2_body(agg3, agg4, h1p, h2p, wr3, wo3, b3, g3, bt3, wr4, wo4, b4, g4,
              bt4, g5, bt5, wl, bl, out):
  h1k = h1p[pl.ds(0, K), :]
  h2k = h2p[pl.ds(0, K), :]
  h3 = _bnrelu(_mm(agg3[...], wr3[...]) + _mm(h1k, wo3[...]) + b3[...], g3[...],
               bt3[...])
  h4 = _bnrelu(_mm(agg4[...], wr4[...]) + _mm(h2k, wo4[...]) + b4[...], g4[...],
               bt4[...])
  rows = []
  for g in range(B):
    blk3 = lax.slice(h3, (g * KP, 0), ((g + 1) * KP, D))
    blk4 = lax.slice(h4, (g * KP, 0), ((g + 1) * KP, D))
    rows.append(jnp.concatenate(
        [jnp.max(blk3, axis=0, keepdims=True),
         jnp.mean(blk3, axis=0, keepdims=True),
         jnp.max(blk4, axis=0, keepdims=True),
         jnp.mean(blk4, axis=0, keepdims=True)], axis=1))
  z = jnp.concatenate(rows, axis=0)
  m = jnp.mean(z, axis=0, keepdims=True)
  v = jnp.mean((z - m) * (z - m), axis=0, keepdims=True)
  z = (z - m) * lax.rsqrt(v + 1e-5) * g5[...] + bt5[...]
  z = _mm(z, wl[...]) + bl[...]
  mx = jnp.max(z, axis=1, keepdims=True)
  ez = jnp.exp(z - mx)
  out[...] = z - mx - jnp.log(jnp.sum(ez, axis=1, keepdims=True))


_tc2 = pl.pallas_call(
    _tc2_body,
    out_shape=jax.ShapeDtypeStruct((B, C), jnp.float32))


def _pad_edges(e):
  s = jnp.concatenate([e[0].astype(jnp.int32),
                       jnp.full((EPAD - E,), N, jnp.int32)])
  d = jnp.concatenate([e[1].astype(jnp.int32),
                       jnp.arange(EPAD - E, dtype=jnp.int32) % N])
  return s, d


def kernel(x, fc_x, edge_index, fc_edge_index, batch, W_rel1, W_root1,
           W_rel2, W_root2, W_rel3, W_root3, W_rel4, W_root4, b1, b2, b3, b4,
           pool_w, g1, g2, g3, g4, bt1, bt2, bt3, bt4, g5, bt5, W_lin,
           b_lin):
  s1, d1 = _pad_edges(edge_index)
  s2, d2 = _pad_edges(fc_edge_index)
  zeros = jnp.zeros((NPAD, D), jnp.float32)
  r = lambda a: a.reshape(1, -1)

  zpad = jnp.zeros((8, D), jnp.float32)
  xpad = jnp.concatenate([x, zpad], axis=0)
  fpad = jnp.concatenate([fc_x, zpad], axis=0)
  agg1, agg2 = _sc_seg1(xpad, fpad, s1, d1, s2, d2, zeros)
  h1s, h2s, ninv2, perm2 = _tc1(
      x, fc_x, agg1, agg2, W_rel1, W_root1, r(b1), r(g1), r(bt1),
      W_rel2, W_root2, r(b2), r(g2), r(bt2), pool_w.reshape(2 * D, 1))
  ninv = ninv2.reshape(NPAD)
  perm = perm2.reshape(K)
  h1p, h2p = _sc_build(h1s, h2s, perm, zeros)
  agg3, agg4 = _sc_seg2(h1p, h2p, s1, d1, s2, d2, ninv, zeros)
  return _tc2(agg3, agg4, h1p, h2p, W_rel3, W_root3, r(b3), r(g3), r(bt3),
              W_rel4, W_root4, r(b4), r(g4), r(bt4), r(g5), r(bt5),
              W_lin, r(b_lin))


# speedup vs baseline: 9.6772x; 9.6772x over previous
import functools

import jax
import jax.numpy as jnp
from jax import lax
from jax.experimental import pallas as pl
from jax.experimental.pallas import tpu as pltpu
from jax.experimental.pallas import tpu_sc as plsc

N = 10000
E = 320000
B = 20
NP = 500
KP = 400
D = 128
C = 2
K = B * KP
EC = 128
NSUB = 16
RPT = 162
ERP = RPT * NSUB
EPAD = ERP * EC
NPAD = N + 8
KPAD = K + 8
ZR = 512
NTAB = N + ZR
KTAB = K + ZR
NBUF = 3

_MESH = plsc.VectorSubcoreMesh(core_axis_name="c", subcore_axis_name="s")


def _run_stream(table, s1d, d1d, acc, sidx, didx, sorig, dorig, rows, gsem,
                ssem, ninv_v, sub):
  base = sub * RPT

  def load_idx(i, p):
    off = (base + i) * EC
    if ninv_v is None:
      pltpu.sync_copy(s1d.at[pl.ds(off, EC)], sidx.at[p])
      pltpu.sync_copy(d1d.at[pl.ds(off, EC)], didx.at[p])
    else:
      pltpu.sync_copy(s1d.at[pl.ds(off, EC)], sorig.at[p])
      pltpu.sync_copy(d1d.at[pl.ds(off, EC)], dorig.at[p])
      lane = lax.iota(jnp.int32, 16)
      for g in range(EC // 16):
        sl = pl.ds(g * 16, 16)
        sv = plsc.load_gather(ninv_v, [sorig[p, sl]])
        dv = plsc.load_gather(ninv_v, [dorig[p, sl]])
        ok = (sv < K) & (dv < K)
        zrow = (K + ((sub * EC + g * 16) & (ZR - 1))) + lane
        dump = (sub * EC + g * 16) * 3 + lane * 3
        sidx[p, sl] = jnp.where(ok, sv, zrow)
        didx[p, sl] = jnp.where(ok, dv, dump)

  def gather(p):
    pltpu.async_copy(table.at[sidx.at[p]], rows.at[p], gsem.at[p])

  def wait_gather(p):
    pltpu.make_async_copy(table.at[sidx.at[p]], rows.at[p], gsem.at[p]).wait()

  def scat(p):
    pltpu.async_copy(rows.at[p], acc.at[didx.at[p]], ssem.at[p], add=True)

  def wait_scat(p):
    pltpu.make_async_copy(rows.at[p], acc.at[didx.at[p]], ssem.at[p]).wait()

  for i in range(NBUF):
    load_idx(i, i)
    gather(i)
  wait_gather(0)
  scat(0)
  wait_gather(1)
  scat(1)

  def body3(j, carry):
    for u in range(3):
      i = 3 * j + 3 + u
      p = u
      q = (u + 2) % 3
      wait_scat(p)
      load_idx(i, p)
      gather(p)
      wait_gather(q)
      scat(q)
    return carry

  lax.fori_loop(0, (RPT - 3) // 3, body3, 0)

  wait_gather((RPT - 1) % 3)
  scat((RPT - 1) % 3)
  for p in range(NBUF):
    wait_scat(p)


def _zero_acc(zeros, acc, sub, per, pad_rows):
  pltpu.sync_copy(zeros.at[pl.ds(sub * per, per)],
                  acc.at[pl.ds(sub * per, per)])
  extra = pad_rows - NSUB * per

  @pl.when(sub == 0)
  def _():
    pltpu.sync_copy(zeros.at[pl.ds(0, extra)],
                    acc.at[pl.ds(NSUB * per, extra)])


def _readout(acc, out, sub, per, nrows):
  pltpu.sync_copy(acc.at[pl.ds(sub * per, per)],
                  out.at[pl.ds(sub * per, per)])
  extra = nrows - NSUB * per

  @pl.when(sub == 0)
  def _():
    pltpu.sync_copy(acc.at[pl.ds(NSUB * per, extra)],
                    out.at[pl.ds(NSUB * per, extra)])


def _seg1_body(x1, x2, s1, d1, s2, d2, zeros, agg1, agg2,
               acc, sidx, didx, rows, gsem, ssem):
  c = lax.axis_index("c")
  s = lax.axis_index("s")
  _zero_acc(zeros, acc, s, 624, NPAD)
  plsc.subcore_barrier()

  @pl.when(c == 0)
  def _():
    _run_stream(x1, s1, d1, acc, sidx, didx, None, None, rows, gsem, ssem,
                None, s)

  @pl.when(c == 1)
  def _():
    _run_stream(x2, s2, d2, acc, sidx, didx, None, None, rows, gsem, ssem,
                None, s)

  plsc.subcore_barrier()

  @pl.when(c == 0)
  def _():
    _readout(acc, agg1, s, 624, N)

  @pl.when(c == 1)
  def _():
    _readout(acc, agg2, s, 624, N)


def _seg2_body(t1, t2, s1, d1, s2, d2, ninv, zeros, agg3, agg4,
               acc, sidx, didx, sorig, dorig, rows, ninv_v, gsem, ssem):
  c = lax.axis_index("c")
  s = lax.axis_index("s")
  pltpu.sync_copy(ninv, ninv_v)
  _zero_acc(zeros, acc, s, 496, KPAD)
  plsc.subcore_barrier()

  @pl.when(c == 0)
  def _():
    _run_stream(t1, s1, d1, acc, sidx, didx, sorig, dorig, rows, gsem, ssem,
                ninv_v, s)

  @pl.when(c == 1)
  def _():
    _run_stream(t2, s2, d2, acc, sidx, didx, sorig, dorig, rows, gsem, ssem,
                ninv_v, s)

  plsc.subcore_barrier()

  @pl.when(c == 0)
  def _():
    _readout(acc, agg3, s, 496, K)

  @pl.when(c == 1)
  def _():
    _readout(acc, agg4, s, 496, K)


_NCHUNK = K // 80


def _build_body(h1s, h2s, perm, zeros, h1p, h2p, pidx, prow, psem):
  c = lax.axis_index("c")
  s = lax.axis_index("s")

  def run(table, outp):
    cs = (s * _NCHUNK) // NSUB
    ce = ((s + 1) * _NCHUNK) // NSUB
    for t in range(_NCHUNK // NSUB + 1):
      idx = cs + t

      @pl.when(idx < ce)
      def _():
        q0 = idx * 80
        pltpu.sync_copy(perm.at[pl.ds(q0, 80)], pidx)
        pltpu.async_copy(table.at[pidx], prow, psem).wait()
        pltpu.sync_copy(prow, outp.at[pl.ds(q0, 80)])

    pltpu.sync_copy(zeros.at[pl.ds(0, ZR // NSUB)],
                    outp.at[pl.ds(K + s * (ZR // NSUB), ZR // NSUB)])

  @pl.when(c == 0)
  def _():
    run(h1s, h1p)

  @pl.when(c == 1)
  def _():
    run(h2s, h2p)


_sc_seg1 = functools.partial(
    pl.kernel, _seg1_body,
    out_type=(jax.ShapeDtypeStruct((N, D), jnp.float32),
              jax.ShapeDtypeStruct((N, D), jnp.float32)),
    mesh=_MESH,
    compiler_params=pltpu.CompilerParams(needs_layout_passes=False),
    scratch_types=(
        pltpu.VMEM_SHARED((NPAD, D), jnp.float32),
        pltpu.VMEM((NBUF, EC), jnp.int32),
        pltpu.VMEM((NBUF, EC), jnp.int32),
        pltpu.VMEM((NBUF, EC, D), jnp.float32),
        pltpu.SemaphoreType.DMA((NBUF,)),
        pltpu.SemaphoreType.DMA((NBUF,)),
    ))()

_sc_seg2 = functools.partial(
    pl.kernel, _seg2_body,
    out_type=(jax.ShapeDtypeStruct((K, D), jnp.float32),
              jax.ShapeDtypeStruct((K, D), jnp.float32)),
    mesh=_MESH,
    compiler_params=pltpu.CompilerParams(needs_layout_passes=False),
    scratch_types=(
        pltpu.VMEM_SHARED((KPAD, D), jnp.float32),
        pltpu.VMEM((NBUF, EC), jnp.int32),
        pltpu.VMEM((NBUF, EC), jnp.int32),
        pltpu.VMEM((NBUF, EC), jnp.int32),
        pltpu.VMEM((NBUF, EC), jnp.int32),
        pltpu.VMEM((NBUF, EC, D), jnp.float32),
        pltpu.VMEM((NTAB,), jnp.int32),
        pltpu.SemaphoreType.DMA((NBUF,)),
        pltpu.SemaphoreType.DMA((NBUF,)),
    ))()

_sc_build = functools.partial(
    pl.kernel, _build_body,
    out_type=(jax.ShapeDtypeStruct((KTAB, D), jnp.float32),
              jax.ShapeDtypeStruct((KTAB, D), jnp.float32)),
    mesh=_MESH,
    compiler_params=pltpu.CompilerParams(needs_layout_passes=False),
    scratch_types=(
        pltpu.VMEM((80,), jnp.int32),
        pltpu.VMEM((80, D), jnp.float32),
        pltpu.SemaphoreType.DMA,
    ))()


_HI = lax.Precision.HIGHEST


def _mm(a, b):
  return jnp.matmul(a, b)


def _bnrelu(t, g, bt):
  m = jnp.mean(t, axis=0, keepdims=True)
  v = jnp.mean((t - m) * (t - m), axis=0, keepdims=True)
  return jnp.maximum((t - m) * lax.rsqrt(v + 1e-5) * g + bt, 0.0)


def _tc1_body(x, fcx, agg1, agg2, wr1, wo1, b1, g1, bt1, wr2, wo2, b2, g2,
              bt2, pw, h1s_o, h2s_o, ninv_o, perm_o):
  xx = x[...]
  ff = fcx[...]
  h1 = _bnrelu(_mm(agg1[...], wr1[...]) + _mm(xx, wo1[...]) + b1[...], g1[...],
               bt1[...])
  h2 = _bnrelu(_mm(agg2[...], wr2[...]) + _mm(ff, wo2[...]) + b2[...], g2[...],
               bt2[...])
  pwv = pw[...]
  score = ((_mm(h1, pwv[:D]) + _mm(h2, pwv[D:]))
           * lax.rsqrt(jnp.sum(pwv * pwv)))
  sc = jnp.tanh(score)
  h1s_o[...] = h1 * sc
  h2s_o[...] = h2 * sc

  ii = lax.broadcasted_iota(jnp.int32, (NP, NP), 0)
  jj = lax.broadcasted_iota(jnp.int32, (NP, NP), 1)
  eye = jnp.where(ii == jj, 1.0, 0.0)
  jlt = jj < ii
  kio = lax.broadcasted_iota(jnp.int32, (KP, NP), 0).astype(jnp.float32)
  icol = lax.broadcasted_iota(jnp.int32, (NP, 1), 0).astype(jnp.float32)
  tdims = (((0,), (0,)), ((), ()))
  hi = lax.Precision.HIGHEST
  for g in range(B):
    sg = lax.slice(score, (g * NP, 0), ((g + 1) * NP, 1))
    sgr = lax.dot_general(sg, eye, tdims, precision=hi)
    beats = (sgr > sg) | ((sgr == sg) & jlt)
    rank = jnp.sum(jnp.where(beats, 1.0, 0.0), axis=1, keepdims=True)
    nn = jnp.where(rank < KP, rank + g * KP, float(K))
    ninv_o[pl.ds(g * NP, NP), :] = nn.astype(jnp.int32)
    rr = lax.dot_general(rank, eye, tdims, precision=hi)
    oh = jnp.where(rr == kio, 1.0, 0.0)
    loc = lax.dot_general(oh, icol, (((1,), (0,)), ((), ())), precision=hi)
    perm_o[pl.ds(g * KP, KP), :] = (loc + g * NP).astype(jnp.int32)
  ninv_o[pl.ds(N, ZR), :] = jnp.full((ZR, 1), K, jnp.int32)


_tc1 = pl.pallas_call(
    _tc1_body,
    out_shape=(jax.ShapeDtypeStruct((N, D), jnp.float32),
               jax.ShapeDtypeStruct((N, D), jnp.float32),
               jax.ShapeDtypeStruct((NTAB, 1), jnp.int32),
               jax.ShapeDtypeStruct((K, 1), jnp.int32)))


def _tc2_body(agg3, agg4, h1p, h2p, wr3, wo3, b3, g3, bt3, wr4, wo4, b4, g4,
              bt4, g5, bt5, wl, bl, out):
  h1k = h1p[pl.ds(0, K), :]
  h2k = h2p[pl.ds(0, K), :]
  h3 = _bnrelu(_mm(agg3[...], wr3[...]) + _mm(h1k, wo3[...]) + b3[...], g3[...],
               bt3[...])
  h4 = _bnrelu(_mm(agg4[...], wr4[...]) + _mm(h2k, wo4[...]) + b4[...], g4[...],
               bt4[...])
  rows = []
  for g in range(B):
    blk3 = lax.slice(h3, (g * KP, 0), ((g + 1) * KP, D))
    blk4 = lax.slice(h4, (g * KP, 0), ((g + 1) * KP, D))
    rows.append(jnp.concatenate(
        [jnp.max(blk3, axis=0, keepdims=True),
         jnp.mean(blk3, axis=0, keepdims=True),
         jnp.max(blk4, axis=0, keepdims=True),
         jnp.mean(blk4, axis=0, keepdims=True)], axis=1))
  z = jnp.concatenate(rows, axis=0)
  m = jnp.mean(z, axis=0, keepdims=True)
  v = jnp.mean((z - m) * (z - m), axis=0, keepdims=True)
  z = (z - m) * lax.rsqrt(v + 1e-5) * g5[...] + bt5[...]
  z = _mm(z, wl[...]) + bl[...]
  mx = jnp.max(z, axis=1, keepdims=True)
  ez = jnp.exp(z - mx)
  out[...] = z - mx - jnp.log(jnp.sum(ez, axis=1, keepdims=True))


_tc2 = pl.pallas_call(
    _tc2_body,
    out_shape=jax.ShapeDtypeStruct((B, C), jnp.float32))


def _pad_edges(e):
  s = jnp.concatenate([e[0].astype(jnp.int32),
                       N + (jnp.arange(EPAD - E, dtype=jnp.int32) % ZR)])
  d = jnp.concatenate([e[1].astype(jnp.int32),
                       jnp.arange(EPAD - E, dtype=jnp.int32) % N])
  return s, d


def kernel(x, fc_x, edge_index, fc_edge_index, batch, W_rel1, W_root1,
           W_rel2, W_root2, W_rel3, W_root3, W_rel4, W_root4, b1, b2, b3, b4,
           pool_w, g1, g2, g3, g4, bt1, bt2, bt3, bt4, g5, bt5, W_lin,
           b_lin):
  s1, d1 = _pad_edges(edge_index)
  s2, d2 = _pad_edges(fc_edge_index)
  zeros = jnp.zeros((NPAD, D), jnp.float32)
  r = lambda a: a.reshape(1, -1)

  zpad = jnp.zeros((ZR, D), jnp.float32)
  xpad = jnp.concatenate([x, zpad], axis=0)
  fpad = jnp.concatenate([fc_x, zpad], axis=0)
  agg1, agg2 = _sc_seg1(xpad, fpad, s1, d1, s2, d2, zeros)
  h1s, h2s, ninv2, perm2 = _tc1(
      x, fc_x, agg1, agg2, W_rel1, W_root1, r(b1), r(g1), r(bt1),
      W_rel2, W_root2, r(b2), r(g2), r(bt2), pool_w.reshape(2 * D, 1))
  ninv = ninv2.reshape(NTAB)
  perm = perm2.reshape(K)
  h1p, h2p = _sc_build(h1s, h2s, perm, zeros)
  agg3, agg4 = _sc_seg2(h1p, h2p, s1, d1, s2, d2, ninv, zeros)
  return _tc2(agg3, agg4, h1p, h2p, W_rel3, W_root3, r(b3), r(g3), r(bt3),
              W_rel4, W_root4, r(b4), r(g4), r(bt4), r(g5), r(bt5),
              W_lin, r(b_lin))


# speedup vs baseline: 9.6872x; 1.0010x over previous
import functools

import jax
import jax.numpy as jnp
from jax import lax
from jax.experimental import pallas as pl
from jax.experimental.pallas import tpu as pltpu
from jax.experimental.pallas import tpu_sc as plsc

N = 10000
E = 320000
B = 20
NP = 500
KP = 400
D = 128
C = 2
K = B * KP
EC = 128
NSUB = 16
RPT = 162
ERP = RPT * NSUB
EPAD = ERP * EC
NPAD = N + 8
KPAD = K + 8
ZR = 512
NTAB = N + ZR
KTAB = K + ZR
NBUF = 3

_MESH = plsc.VectorSubcoreMesh(core_axis_name="c", subcore_axis_name="s")


def _run_stream(table, s1d, d1d, acc, sidx, didx, sorig, dorig, rows, gsem,
                ssem, ninv_v, sub):
  base = sub * RPT
  lane = lax.iota(jnp.int32, 16)

  def load_idx(i, p):
    off = (base + i) * EC
    if ninv_v is None:
      pltpu.sync_copy(s1d.at[pl.ds(off, EC)], sidx.at[p])
      pltpu.sync_copy(d1d.at[pl.ds(off, EC)], didx.at[p])
    else:
      pltpu.sync_copy(s1d.at[pl.ds(off, EC)], sorig.at[p])
      pltpu.sync_copy(d1d.at[pl.ds(off, EC)], dorig.at[p])
      for g in range(EC // 16):
        sl = pl.ds(g * 16, 16)
        sv = plsc.load_gather(ninv_v, [sorig[p, sl]])
        dv = plsc.load_gather(ninv_v, [dorig[p, sl]])
        ok = (sv < K) & (dv < K)
        zrow = (K + ((sub * EC + g * 16) & (ZR - 1))) + lane
        dump = (sub * EC + g * 16) * 3 + lane * 3
        sidx[p, sl] = jnp.where(ok, sv, zrow)
        didx[p, sl] = jnp.where(ok, dv, dump)

  def gather(p):
    pltpu.async_copy(table.at[sidx.at[p]], rows.at[p], gsem.at[p])

  def wait_gather(p):
    pltpu.make_async_copy(table.at[sidx.at[p]], rows.at[p], gsem.at[p]).wait()

  def scat(p):
    pltpu.async_copy(rows.at[p], acc.at[didx.at[p]], ssem.at[p], add=True)

  def wait_scat(p):
    pltpu.make_async_copy(rows.at[p], acc.at[didx.at[p]], ssem.at[p]).wait()

  for i in range(NBUF):
    load_idx(i, i)
    gather(i)
  wait_gather(0)
  scat(0)
  wait_gather(1)
  scat(1)

  def body3(j, carry):
    for u in range(3):
      i = 3 * j + 3 + u
      p = u
      q = (u + 2) % 3
      wait_scat(p)
      load_idx(i, p)
      gather(p)
      wait_gather(q)
      scat(q)
    return carry

  lax.fori_loop(0, (RPT - 3) // 3, body3, 0)

  wait_gather((RPT - 1) % 3)
  scat((RPT - 1) % 3)
  for p in range(NBUF):
    wait_scat(p)


def _zero_acc(zeros, acc, sub, per, pad_rows):
  pltpu.sync_copy(zeros.at[pl.ds(sub * per, per)],
                  acc.at[pl.ds(sub * per, per)])
  extra = pad_rows - NSUB * per

  @pl.when(sub == 0)
  def _():
    pltpu.sync_copy(zeros.at[pl.ds(0, extra)],
                    acc.at[pl.ds(NSUB * per, extra)])


def _readout(acc, out, sub, per, nrows):
  pltpu.sync_copy(acc.at[pl.ds(sub * per, per)],
                  out.at[pl.ds(sub * per, per)])
  extra = nrows - NSUB * per

  @pl.when(sub == 0)
  def _():
    pltpu.sync_copy(acc.at[pl.ds(NSUB * per, extra)],
                    out.at[pl.ds(NSUB * per, extra)])


def _seg1_body(x1, x2, s1, d1, s2, d2, zeros, agg1, agg2,
               acc, sidx, didx, rows, gsem, ssem):
  c = lax.axis_index("c")
  s = lax.axis_index("s")
  _zero_acc(zeros, acc, s, 624, NPAD)
  plsc.subcore_barrier()

  @pl.when(c == 0)
  def _():
    _run_stream(x1, s1, d1, acc, sidx, didx, None, None, rows, gsem, ssem,
                None, s)

  @pl.when(c == 1)
  def _():
    _run_stream(x2, s2, d2, acc, sidx, didx, None, None, rows, gsem, ssem,
                None, s)

  plsc.subcore_barrier()

  @pl.when(c == 0)
  def _():
    _readout(acc, agg1, s, 624, N)

  @pl.when(c == 1)
  def _():
    _readout(acc, agg2, s, 624, N)


def _seg2_body(t1, t2, s1, d1, s2, d2, ninv, zeros, agg3, agg4,
               acc, sidx, didx, sorig, dorig, rows, ninv_v, gsem, ssem):
  c = lax.axis_index("c")
  s = lax.axis_index("s")
  pltpu.sync_copy(ninv, ninv_v)
  _zero_acc(zeros, acc, s, 496, KPAD)
  plsc.subcore_barrier()

  @pl.when(c == 0)
  def _():
    _run_stream(t1, s1, d1, acc, sidx, didx, sorig, dorig, rows, gsem, ssem,
                ninv_v, s)

  @pl.when(c == 1)
  def _():
    _run_stream(t2, s2, d2, acc, sidx, didx, sorig, dorig, rows, gsem, ssem,
                ninv_v, s)

  plsc.subcore_barrier()

  @pl.when(c == 0)
  def _():
    _readout(acc, agg3, s, 496, K)

  @pl.when(c == 1)
  def _():
    _readout(acc, agg4, s, 496, K)


_NCHUNK = K // 80


def _build_body(h1s, h2s, perm, zeros, h1p, h2p, pidx, prow, psem):
  c = lax.axis_index("c")
  s = lax.axis_index("s")

  def run(table, outp):
    cs = (s * _NCHUNK) // NSUB
    ce = ((s + 1) * _NCHUNK) // NSUB
    for t in range(_NCHUNK // NSUB + 1):
      idx = cs + t

      @pl.when(idx < ce)
      def _():
        q0 = idx * 80
        pltpu.sync_copy(perm.at[pl.ds(q0, 80)], pidx)
        pltpu.async_copy(table.at[pidx], prow, psem).wait()
        pltpu.sync_copy(prow, outp.at[pl.ds(q0, 80)])

    pltpu.sync_copy(zeros.at[pl.ds(0, ZR // NSUB)],
                    outp.at[pl.ds(K + s * (ZR // NSUB), ZR // NSUB)])

  @pl.when(c == 0)
  def _():
    run(h1s, h1p)

  @pl.when(c == 1)
  def _():
    run(h2s, h2p)


_sc_seg1 = functools.partial(
    pl.kernel, _seg1_body,
    out_type=(jax.ShapeDtypeStruct((N, D), jnp.float32),
              jax.ShapeDtypeStruct((N, D), jnp.float32)),
    mesh=_MESH,
    compiler_params=pltpu.CompilerParams(needs_layout_passes=False),
    scratch_types=(
        pltpu.VMEM_SHARED((NPAD, D), jnp.float32),
        pltpu.VMEM((NBUF, EC), jnp.int32),
        pltpu.VMEM((NBUF, EC), jnp.int32),
        pltpu.VMEM((NBUF, EC, D), jnp.float32),
        pltpu.SemaphoreType.DMA((NBUF,)),
        pltpu.SemaphoreType.DMA((NBUF,)),
    ))()

_sc_seg2 = functools.partial(
    pl.kernel, _seg2_body,
    out_type=(jax.ShapeDtypeStruct((K, D), jnp.float32),
              jax.ShapeDtypeStruct((K, D), jnp.float32)),
    mesh=_MESH,
    compiler_params=pltpu.CompilerParams(needs_layout_passes=False),
    scratch_types=(
        pltpu.VMEM_SHARED((KPAD, D), jnp.float32),
        pltpu.VMEM((NBUF, EC), jnp.int32),
        pltpu.VMEM((NBUF, EC), jnp.int32),
        pltpu.VMEM((NBUF, EC), jnp.int32),
        pltpu.VMEM((NBUF, EC), jnp.int32),
        pltpu.VMEM((NBUF, EC, D), jnp.float32),
        pltpu.VMEM((NTAB,), jnp.int32),
        pltpu.SemaphoreType.DMA((NBUF,)),
        pltpu.SemaphoreType.DMA((NBUF,)),
    ))()

_sc_build = functools.partial(
    pl.kernel, _build_body,
    out_type=(jax.ShapeDtypeStruct((KTAB, D), jnp.float32),
              jax.ShapeDtypeStruct((KTAB, D), jnp.float32)),
    mesh=_MESH,
    compiler_params=pltpu.CompilerParams(needs_layout_passes=False),
    scratch_types=(
        pltpu.VMEM((80,), jnp.int32),
        pltpu.VMEM((80, D), jnp.float32),
        pltpu.SemaphoreType.DMA,
    ))()


_HI = lax.Precision.HIGHEST


def _mm(a, b):
  return jnp.matmul(a, b)


def _bnrelu(t, g, bt):
  m = jnp.mean(t, axis=0, keepdims=True)
  v = jnp.mean((t - m) * (t - m), axis=0, keepdims=True)
  return jnp.maximum((t - m) * lax.rsqrt(v + 1e-5) * g + bt, 0.0)


def _tc1_body(x, fcx, agg1, agg2, wr1, wo1, b1, g1, bt1, wr2, wo2, b2, g2,
              bt2, pw, h1s_o, h2s_o, ninv_o, perm_o):
  xx = x[...]
  ff = fcx[...]
  h1 = _bnrelu(_mm(agg1[...], wr1[...]) + _mm(xx, wo1[...]) + b1[...], g1[...],
               bt1[...])
  h2 = _bnrelu(_mm(agg2[...], wr2[...]) + _mm(ff, wo2[...]) + b2[...], g2[...],
               bt2[...])
  pwv = pw[...]
  score = ((_mm(h1, pwv[:D]) + _mm(h2, pwv[D:]))
           * lax.rsqrt(jnp.sum(pwv * pwv)))
  sc = jnp.tanh(score)
  h1s_o[...] = h1 * sc
  h2s_o[...] = h2 * sc

  ii = lax.broadcasted_iota(jnp.int32, (NP, NP), 0)
  jj = lax.broadcasted_iota(jnp.int32, (NP, NP), 1)
  eye = jnp.where(ii == jj, 1.0, 0.0)
  jlt = jj < ii
  kio = lax.broadcasted_iota(jnp.int32, (KP, NP), 0).astype(jnp.float32)
  icol = lax.broadcasted_iota(jnp.int32, (NP, 1), 0).astype(jnp.float32)
  tdims = (((0,), (0,)), ((), ()))
  hi = lax.Precision.HIGHEST
  for g in range(B):
    sg = lax.slice(score, (g * NP, 0), ((g + 1) * NP, 1))
    sgr = lax.dot_general(sg, eye, tdims, precision=hi)
    beats = (sgr > sg) | ((sgr == sg) & jlt)
    rank = jnp.sum(jnp.where(beats, 1.0, 0.0), axis=1, keepdims=True)
    nn = jnp.where(rank < KP, rank + g * KP, float(K))
    ninv_o[pl.ds(g * NP, NP), :] = nn.astype(jnp.int32)
    rr = lax.dot_general(rank, eye, tdims, precision=hi)
    oh = jnp.where(rr == kio, 1.0, 0.0)
    loc = lax.dot_general(oh, icol, (((1,), (0,)), ((), ())), precision=hi)
    perm_o[pl.ds(g * KP, KP), :] = (loc + g * NP).astype(jnp.int32)
  ninv_o[pl.ds(N, ZR), :] = jnp.full((ZR, 1), K, jnp.int32)


_tc1 = pl.pallas_call(
    _tc1_body,
    out_shape=(jax.ShapeDtypeStruct((N, D), jnp.float32),
               jax.ShapeDtypeStruct((N, D), jnp.float32),
               jax.ShapeDtypeStruct((NTAB, 1), jnp.int32),
               jax.ShapeDtypeStruct((K, 1), jnp.int32)))


def _tc2_body(agg3, agg4, h1p, h2p, wr3, wo3, b3, g3, bt3, wr4, wo4, b4, g4,
              bt4, g5, bt5, wl, bl, out):
  h1k = h1p[pl.ds(0, K), :]
  h2k = h2p[pl.ds(0, K), :]
  h3 = _bnrelu(_mm(agg3[...], wr3[...]) + _mm(h1k, wo3[...]) + b3[...], g3[...],
               bt3[...])
  h4 = _bnrelu(_mm(agg4[...], wr4[...]) + _mm(h2k, wo4[...]) + b4[...], g4[...],
               bt4[...])
  rows = []
  for g in range(B):
    blk3 = lax.slice(h3, (g * KP, 0), ((g + 1) * KP, D))
    blk4 = lax.slice(h4, (g * KP, 0), ((g + 1) * KP, D))
    rows.append(jnp.concatenate(
        [jnp.max(blk3, axis=0, keepdims=True),
         jnp.mean(blk3, axis=0, keepdims=True),
         jnp.max(blk4, axis=0, keepdims=True),
         jnp.mean(blk4, axis=0, keepdims=True)], axis=1))
  z = jnp.concatenate(rows, axis=0)
  m = jnp.mean(z, axis=0, keepdims=True)
  v = jnp.mean((z - m) * (z - m), axis=0, keepdims=True)
  z = (z - m) * lax.rsqrt(v + 1e-5) * g5[...] + bt5[...]
  z = _mm(z, wl[...]) + bl[...]
  mx = jnp.max(z, axis=1, keepdims=True)
  ez = jnp.exp(z - mx)
  out[...] = z - mx - jnp.log(jnp.sum(ez, axis=1, keepdims=True))


_tc2 = pl.pallas_call(
    _tc2_body,
    out_shape=jax.ShapeDtypeStruct((B, C), jnp.float32))


def _pad_edges(e):
  s = jnp.concatenate([e[0].astype(jnp.int32),
                       N + (jnp.arange(EPAD - E, dtype=jnp.int32) % ZR)])
  d = jnp.concatenate([e[1].astype(jnp.int32),
                       jnp.arange(EPAD - E, dtype=jnp.int32) % N])
  return s, d


def kernel(x, fc_x, edge_index, fc_edge_index, batch, W_rel1, W_root1,
           W_rel2, W_root2, W_rel3, W_root3, W_rel4, W_root4, b1, b2, b3, b4,
           pool_w, g1, g2, g3, g4, bt1, bt2, bt3, bt4, g5, bt5, W_lin,
           b_lin):
  s1, d1 = _pad_edges(edge_index)
  s2, d2 = _pad_edges(fc_edge_index)
  zeros = jnp.zeros((NPAD, D), jnp.float32)
  r = lambda a: a.reshape(1, -1)

  zpad = jnp.zeros((ZR, D), jnp.float32)
  xpad = jnp.concatenate([x, zpad], axis=0)
  fpad = jnp.concatenate([fc_x, zpad], axis=0)
  agg1, agg2 = _sc_seg1(xpad, fpad, s1, d1, s2, d2, zeros)
  h1s, h2s, ninv2, perm2 = _tc1(
      x, fc_x, agg1, agg2, W_rel1, W_root1, r(b1), r(g1), r(bt1),
      W_rel2, W_root2, r(b2), r(g2), r(bt2), pool_w.reshape(2 * D, 1))
  ninv = ninv2.reshape(NTAB)
  perm = perm2.reshape(K)
  h1p, h2p = _sc_build(h1s, h2s, perm, zeros)
  agg3, agg4 = _sc_seg2(h1p, h2p, s1, d1, s2, d2, ninv, zeros)
  return _tc2(agg3, agg4, h1p, h2p, W_rel3, W_root3, r(b3), r(g3), r(bt3),
              W_rel4, W_root4, r(b4), r(g4), r(bt4), r(g5), r(bt5),
              W_lin, r(b_lin))
